# baseline (speedup 1.0000x reference)
import jax
import jax.numpy as jnp
from jax import lax
from jax.experimental import pallas as pl
from jax.experimental.pallas import tpu as pltpu
from jax.experimental.pallas import tpu_sc as plsc

_OUT = 192
_H = 224
_W = 224
_WIN = 200
_NC = 2
_NS = 16
_NW = _NC * _NS
_LANES = 16
_GROUPS = _OUT // _LANES


def _sc_body(in_hbm, i_hbm, j_hbm, out_hbm, ij_buf, in_buf, out_buf):
    N, C, H, W = in_hbm.shape
    per_w = (N * C) // _NW
    wid = lax.axis_index("s") * _NC + lax.axis_index("c")
    pltpu.sync_copy(i_hbm, ij_buf.at[0])
    pltpu.sync_copy(j_hbm, ij_buf.at[1])
    lane = lax.iota(jnp.int32, _LANES)

    def pair_body(p, carry):
        pair = wid * per_w + p
        n = pair // C
        cc = pair % C
        sel = lane == n
        i0 = jnp.sum(jnp.where(sel, ij_buf[0, :], 0))
        j0 = jnp.sum(jnp.where(sel, ij_buf[1, :], 0))
        ibase = pl.multiple_of((i0 // 8) * 8, 8)
        rsub = i0 - ibase
        pltpu.sync_copy(in_hbm.at[n, cc, pl.ds(ibase, _WIN), :], in_buf)

        cols = [g * _LANES + j0 + lane for g in range(_GROUPS)]

        @plsc.parallel_loop(0, _OUT, step=1, unroll=4)
        def row_body(row):
            rowvec = jnp.full((_LANES,), row + rsub, jnp.int32)
            for g in range(_GROUPS):
                v = plsc.load_gather(in_buf, [rowvec, cols[g]])
                out_buf[row, pl.ds(g * _LANES, _LANES)] = v
        pltpu.sync_copy(out_buf, out_hbm.at[n, cc])
        return carry

    lax.fori_loop(0, per_w, pair_body, 0)


def kernel(tensor, i, j):
    N, C, H, W = tensor.shape
    mesh = plsc.VectorSubcoreMesh(core_axis_name="c", subcore_axis_name="s")
    run = pl.kernel(
        _sc_body,
        out_type=jax.ShapeDtypeStruct((N, C, _OUT, _OUT), tensor.dtype),
        mesh=mesh,
        scratch_types=[
            pltpu.VMEM((2, _LANES), jnp.int32),
            pltpu.VMEM((_WIN, _W), jnp.float32),
            pltpu.VMEM((_OUT, _OUT), jnp.float32),
        ],
        compiler_params=pltpu.CompilerParams(
            use_tc_tiling_on_sc=True, needs_layout_passes=False),
    )
    return run(tensor, i, j)


# speedup vs baseline: 1.4548x; 1.4548x over previous
import jax
import jax.numpy as jnp
from jax import lax
from jax.experimental import pallas as pl
from jax.experimental.pallas import tpu as pltpu
from jax.experimental.pallas import tpu_sc as plsc

_OUT = 192
_HALF = 96
_WIN = _HALF + 8
_H = 224
_W = 224
_NC = 2
_NS = 16
_NW = _NC * _NS
_LANES = 16
_GROUPS = _OUT // _LANES


def _sc_body(in_hbm, i_hbm, j_hbm, out_hbm,
             ij_buf, in_buf0, in_buf1, out_buf0, out_buf1,
             isem0, isem1, osem0, osem1):
    N, C, H, W = in_hbm.shape
    items = (N * C * 2) // _NW
    wid = lax.axis_index("s") * _NC + lax.axis_index("c")
    base = wid * items
    pltpu.sync_copy(i_hbm, ij_buf.at[0])
    pltpu.sync_copy(j_hbm, ij_buf.at[1])
    lane = lax.iota(jnp.int32, _LANES)

    def params(t):
        item = base + t
        pair = item // 2
        h = item % 2
        n = pair // C
        cc = pair % C
        sel = lane == n
        i0 = jnp.sum(jnp.where(sel, ij_buf[0, :], 0))
        j0 = jnp.sum(jnp.where(sel, ij_buf[1, :], 0))
        row0 = i0 + h * _HALF
        ibase = pl.multiple_of(jnp.minimum((row0 // 8) * 8, H - _WIN), 8)
        rsub = row0 - ibase
        return n, cc, h, ibase, rsub, j0

    def start_in(t, buf, sem):
        n, cc, h, ibase, rsub, j0 = params(t)
        pltpu.async_copy(in_hbm.at[n, cc, pl.ds(ibase, _WIN), :], buf, sem)

    def wait_in(buf, sem):
        pltpu.make_async_copy(in_hbm.at[0, 0, pl.ds(0, _WIN), :], buf, sem).wait()

    def start_out(t, buf, sem):
        n, cc, h, _, _, _ = params(t)
        pltpu.async_copy(
            buf, out_hbm.at[n, cc, pl.ds(h * _HALF, _HALF), :], sem)

    def wait_out(buf, sem):
        pltpu.make_async_copy(
            buf, out_hbm.at[0, 0, pl.ds(0, _HALF), :], sem).wait()

    def compute(t, in_buf, out_buf):
        _, _, _, _, rsub, j0 = params(t)
        cols = [g * _LANES + j0 + lane for g in range(_GROUPS)]

        @plsc.parallel_loop(0, _HALF, step=1, unroll=2)
        def row_body(row):
            rowvec = jnp.full((_LANES,), row + rsub, jnp.int32)
            for g in range(_GROUPS):
                v = plsc.load_gather(in_buf, [rowvec, cols[g]])
                out_buf[row, pl.ds(g * _LANES, _LANES)] = v

    start_in(0, in_buf0, isem0)
    nq = items // 2

    def q_body(q, carry):
        t0 = 2 * q
        t1 = t0 + 1
        start_in(t1, in_buf1, isem1)
        wait_in(in_buf0, isem0)

        @pl.when(q > 0)
        def _():
            wait_out(out_buf0, osem0)

        compute(t0, in_buf0, out_buf0)
        start_out(t0, out_buf0, osem0)

        @pl.when(q < nq - 1)
        def _():
            start_in(t0 + 2, in_buf0, isem0)

        wait_in(in_buf1, isem1)

        @pl.when(q > 0)
        def _():
            wait_out(out_buf1, osem1)

        compute(t1, in_buf1, out_buf1)
        start_out(t1, out_buf1, osem1)
        return carry

    lax.fori_loop(0, nq, q_body, 0)
    wait_out(out_buf0, osem0)
    wait_out(out_buf1, osem1)


def kernel(tensor, i, j):
    N, C, H, W = tensor.shape
    mesh = plsc.VectorSubcoreMesh(core_axis_name="c", subcore_axis_name="s")
    run = pl.kernel(
        _sc_body,
        out_type=jax.ShapeDtypeStruct((N, C, _OUT, _OUT), tensor.dtype),
        mesh=mesh,
        scratch_types=[
            pltpu.VMEM((2, _LANES), jnp.int32),
            pltpu.VMEM((_WIN, _W), jnp.float32),
            pltpu.VMEM((_WIN, _W), jnp.float32),
            pltpu.VMEM((_HALF, _OUT), jnp.float32),
            pltpu.VMEM((_HALF, _OUT), jnp.float32),
            pltpu.SemaphoreType.DMA,
            pltpu.SemaphoreType.DMA,
            pltpu.SemaphoreType.DMA,
            pltpu.SemaphoreType.DMA,
        ],
        compiler_params=pltpu.CompilerParams(
            use_tc_tiling_on_sc=True, needs_layout_passes=False),
    )
    return run(tensor, i, j)
